# baseline (speedup 1.0000x reference)
import jax
import jax.numpy as jnp
from jax import lax
from jax.experimental import pallas as pl
from jax.experimental.pallas import tpu as pltpu


def _vq_body(zt_ref, cbt_ref, out_ref):
    ztb = zt_ref[0]
    cbt = cbt_ref[...]
    zb = ztb.T
    zn = jnp.sum(zb * zb, axis=1, keepdims=True)
    cn = jnp.sum(cbt * cbt, axis=0, keepdims=True)
    d2 = zn - 2.0 * jnp.dot(zb, cbt, preferred_element_type=jnp.float32) + cn
    d2 = jnp.maximum(d2, 0.0)
    m = jnp.min(d2, axis=1, keepdims=True)
    iota = lax.broadcasted_iota(jnp.int32, d2.shape, 1)
    idx = jnp.min(jnp.where(d2 == m, iota, jnp.int32(1 << 30)),
                  axis=1, keepdims=True)
    onehot = (iota == idx).astype(jnp.float32)
    res = lax.dot_general(onehot, cbt, (((1,), (1,)), ((), ())),
                          precision=lax.Precision.HIGHEST,
                          preferred_element_type=jnp.float32)
    out_ref[0] = res.T


def _vq_tc(zt, cbt, interpret=False):
    nb, d, b = zt.shape
    k = cbt.shape[1]
    return pl.pallas_call(
        _vq_body,
        grid=(nb,),
        in_specs=[
            pl.BlockSpec((1, d, b), lambda i: (i, 0, 0)),
            pl.BlockSpec((d, k), lambda i: (0, 0)),
        ],
        out_specs=pl.BlockSpec((1, d, b), lambda i: (i, 0, 0)),
        out_shape=jax.ShapeDtypeStruct((nb, d, b), jnp.float32),
        interpret=interpret,
    )(zt, cbt)


def kernel(z, codebook):
    zt = jnp.swapaxes(z, 1, 2)
    out_t = _vq_tc(zt, codebook.T)
    return jnp.swapaxes(out_t, 1, 2)


# speedup vs baseline: 1.4512x; 1.4512x over previous
import jax
import jax.numpy as jnp
from jax import lax
from jax.experimental import pallas as pl
from jax.experimental.pallas import tpu as pltpu


def _vq_body(zt_ref, cbt_ref, out_ref):
    ztb = zt_ref[0]
    cbt = cbt_ref[...]
    zn = jnp.sum(ztb * ztb, axis=0, keepdims=True)
    cn = jnp.sum(cbt * cbt, axis=0, keepdims=True).T
    dot = lax.dot_general(cbt, ztb, (((0,), (0,)), ((), ())),
                          preferred_element_type=jnp.float32)
    d2 = cn - 2.0 * dot + zn
    d2 = jnp.maximum(d2, 0.0)
    m = jnp.min(d2, axis=0, keepdims=True)
    iota = lax.broadcasted_iota(jnp.int32, d2.shape, 0)
    idx = jnp.min(jnp.where(d2 == m, iota, jnp.int32(1 << 30)),
                  axis=0, keepdims=True)
    onehot = (iota == idx).astype(jnp.float32)
    out_ref[0] = lax.dot_general(cbt, onehot, (((1,), (0,)), ((), ())),
                                 preferred_element_type=jnp.float32)


def _vq_tc(zt, cbt, interpret=False):
    nb, d, b = zt.shape
    k = cbt.shape[1]
    return pl.pallas_call(
        _vq_body,
        grid=(nb,),
        in_specs=[
            pl.BlockSpec((1, d, b), lambda i: (i, 0, 0)),
            pl.BlockSpec((d, k), lambda i: (0, 0)),
        ],
        out_specs=pl.BlockSpec((1, d, b), lambda i: (i, 0, 0)),
        out_shape=jax.ShapeDtypeStruct((nb, d, b), jnp.float32),
        interpret=interpret,
    )(zt, cbt)


def kernel(z, codebook):
    zt = jnp.swapaxes(z, 1, 2)
    out_t = _vq_tc(zt, codebook.T)
    return jnp.swapaxes(out_t, 1, 2)


# speedup vs baseline: 1.6245x; 1.1194x over previous
import jax
import jax.numpy as jnp
from jax import lax
from jax.experimental import pallas as pl
from jax.experimental.pallas import tpu as pltpu

_KC = 128


def _vq_body(zt_ref, cbt_ref, out_ref):
    ztb = zt_ref[0]
    cbt = cbt_ref[...]
    bz = ztb.shape[1]
    k = cbt.shape[1]
    zn = jnp.sum(ztb * ztb, axis=0, keepdims=True)
    cn = jnp.sum(cbt * cbt, axis=0, keepdims=True).T
    ztb2 = -2.0 * ztb
    iota = lax.broadcasted_iota(jnp.int32, (_KC, bz), 0)

    m_run = jnp.full((1, bz), jnp.inf, jnp.float32)
    idx_run = jnp.zeros((1, bz), jnp.int32)
    for c in range(k // _KC):
        cbt_c = cbt[:, c * _KC:(c + 1) * _KC]
        cn_c = cn[c * _KC:(c + 1) * _KC]
        dot_c = lax.dot_general(cbt_c, ztb2, (((0,), (0,)), ((), ())),
                                preferred_element_type=jnp.float32)
        d2 = cn_c + dot_c + zn
        m_c = jnp.min(d2, axis=0, keepdims=True)
        idx_c = jnp.min(jnp.where(d2 == m_c, iota, jnp.int32(1 << 30)),
                        axis=0, keepdims=True) + jnp.int32(c * _KC)
        better = m_c < m_run
        idx_run = jnp.where(better, idx_c, idx_run)
        m_run = jnp.minimum(m_run, m_c)

    acc = jnp.zeros((cbt.shape[0], bz), jnp.float32)
    for c in range(k // _KC):
        cbt_c = cbt[:, c * _KC:(c + 1) * _KC]
        onehot = (iota == idx_run - jnp.int32(c * _KC)).astype(jnp.float32)
        acc = acc + lax.dot_general(cbt_c, onehot, (((1,), (0,)), ((), ())),
                                    preferred_element_type=jnp.float32)
    out_ref[0] = acc


def _vq_tc(zt, cbt, interpret=False):
    nb, d, b = zt.shape
    k = cbt.shape[1]
    return pl.pallas_call(
        _vq_body,
        grid=(nb,),
        in_specs=[
            pl.BlockSpec((1, d, b), lambda i: (i, 0, 0)),
            pl.BlockSpec((d, k), lambda i: (0, 0)),
        ],
        out_specs=pl.BlockSpec((1, d, b), lambda i: (i, 0, 0)),
        out_shape=jax.ShapeDtypeStruct((nb, d, b), jnp.float32),
        interpret=interpret,
    )(zt, cbt)


def kernel(z, codebook):
    zt = jnp.swapaxes(z, 1, 2)
    out_t = _vq_tc(zt, codebook.T)
    return jnp.swapaxes(out_t, 1, 2)


# speedup vs baseline: 2.5328x; 1.5591x over previous
import jax
import jax.numpy as jnp
from jax import lax
from jax.experimental import pallas as pl

_KC = 128


def _vq_body(zt_ref, cbt_ref, out_ref):
    cbt = cbt_ref[...]
    k = cbt.shape[1]
    nb = zt_ref.shape[0]
    bz = zt_ref.shape[2]
    cn = jnp.sum(cbt * cbt, axis=0, keepdims=True).T
    iota = lax.broadcasted_iota(jnp.int32, (_KC, bz), 0)

    for b in range(nb):
        ztb = zt_ref[b]
        zn = jnp.sum(ztb * ztb, axis=0, keepdims=True)
        ztb2 = -2.0 * ztb
        m_run = jnp.full((1, bz), jnp.inf, jnp.float32)
        idx_run = jnp.zeros((1, bz), jnp.int32)
        for c in range(k // _KC):
            cbt_c = cbt[:, c * _KC:(c + 1) * _KC]
            cn_c = cn[c * _KC:(c + 1) * _KC]
            dot_c = lax.dot_general(cbt_c, ztb2, (((0,), (0,)), ((), ())),
                                    preferred_element_type=jnp.float32)
            d2 = cn_c + dot_c + zn
            m_c = jnp.min(d2, axis=0, keepdims=True)
            idx_c = jnp.min(jnp.where(d2 == m_c, iota, jnp.int32(1 << 30)),
                            axis=0, keepdims=True) + jnp.int32(c * _KC)
            better = m_c < m_run
            idx_run = jnp.where(better, idx_c, idx_run)
            m_run = jnp.minimum(m_run, m_c)
        acc = jnp.zeros((cbt.shape[0], bz), jnp.float32)
        for c in range(k // _KC):
            cbt_c = cbt[:, c * _KC:(c + 1) * _KC]
            onehot = (iota == idx_run - jnp.int32(c * _KC)).astype(jnp.float32)
            acc = acc + lax.dot_general(cbt_c, onehot, (((1,), (0,)), ((), ())),
                                        preferred_element_type=jnp.float32)
        out_ref[b] = acc


def _vq_tc(zt, cbt, interpret=False):
    nb, d, b = zt.shape
    return pl.pallas_call(
        _vq_body,
        out_shape=jax.ShapeDtypeStruct((nb, d, b), jnp.float32),
        interpret=interpret,
    )(zt, cbt)


def kernel(z, codebook):
    zt = jnp.swapaxes(z, 1, 2)
    out_t = _vq_tc(zt, codebook.T)
    return jnp.swapaxes(out_t, 1, 2)
